# Initial kernel scaffold; baseline (speedup 1.0000x reference)
import jax
import jax.numpy as jnp
from jax.experimental import pallas as pl


def kernel(e, x, feat_table, name_emb):
    raise NotImplementedError("write your pallas kernel here")


# speedup vs baseline: 7.2194x; 7.2194x over previous
import functools

import jax
import jax.numpy as jnp
from jax import lax
from jax.experimental import pallas as pl
from jax.experimental.pallas import tpu as pltpu
from jax.experimental.pallas import tpu_sc as plsc

NUM_USERS = 200000
LANES = 16
CHUNK = 128


@functools.cache
def _build(B, D, XW, NC, NS):
    NW = NC * NS
    b_per_w = B // NW
    n_chunks = b_per_w // CHUNK
    n_groups = b_per_w // LANES
    mesh = plsc.VectorSubcoreMesh(core_axis_name="c", subcore_axis_name="s")

    @functools.partial(
        pl.kernel,
        mesh=mesh,
        compiler_params=pltpu.CompilerParams(
            needs_layout_passes=False, use_tc_tiling_on_sc=False),
        out_type=jax.ShapeDtypeStruct((B, D), jnp.float32),
        scratch_types=[
            pltpu.VMEM((b_per_w,), jnp.int32),
            pltpu.VMEM((4, b_per_w), jnp.int32),
            pltpu.VMEM((4, b_per_w), jnp.int32),
            pltpu.VMEM((b_per_w, D), jnp.float32),
            pltpu.VMEM((68, D), jnp.float32),
            pltpu.SemaphoreType.DMA,
        ],
    )
    def sc_kernel(e_hbm, x_hbm, ft_hbm, name_hbm, out_hbm,
                  e_v, xi_v, xc_v, acc_v, ft_v, sem):
        wid = lax.axis_index("s") * NC + lax.axis_index("c")
        base = wid * b_per_w

        pltpu.sync_copy(e_hbm.at[wid], e_v)
        pltpu.sync_copy(ft_hbm, ft_v)

        descs = []
        for k in range(n_chunks):
            descs.append(pltpu.async_copy(
                name_hbm.at[e_v.at[pl.ds(k * CHUNK, CHUNK)]],
                acc_v.at[pl.ds(k * CHUNK, CHUNK)], sem))

        def idx_body(j, carry):
            ev7 = e_v[pl.ds(j * LANES, LANES)] * 7
            for c in range(4):
                xi_v[c, pl.ds(j * LANES, LANES)] = ev7 + (3 + c)
            return carry

        lax.fori_loop(0, n_groups, idx_body, 0)

        for c in range(4):
            for k in range(n_chunks):
                descs.append(pltpu.async_copy(
                    x_hbm.at[xi_v.at[c, pl.ds(k * CHUNK, CHUNK)]],
                    xc_v.at[c, pl.ds(k * CHUNK, CHUNK)], sem))
        for dsc in descs:
            dsc.wait()

        iota = lax.iota(jnp.int32, LANES)
        zeros = jnp.zeros((LANES,), jnp.int32)

        def group(g, carry):
            row16 = g * LANES + iota
            ev = e_v[pl.ds(g * LANES, LANES)]
            mask = ev >= NUM_USERS
            f3 = xc_v[0, pl.ds(g * LANES, LANES)] + 46
            f4 = xc_v[1, pl.ds(g * LANES, LANES)] + 35
            f5 = xc_v[2, pl.ds(g * LANES, LANES)]
            f6 = xc_v[3, pl.ds(g * LANES, LANES)] + 9
            for d in range(D):
                col = zeros + d
                s = (plsc.load_gather(ft_v, [f5, col])
                     + plsc.load_gather(ft_v, [f6, col])
                     + plsc.load_gather(ft_v, [f4, col])
                     + plsc.load_gather(ft_v, [f3, col]))
                plsc.addupdate_scatter(acc_v, [row16, col], s, mask=mask)
            return carry

        lax.fori_loop(0, n_groups, group, 0)
        pltpu.sync_copy(acc_v, out_hbm.at[pl.ds(base, b_per_w)])

    return sc_kernel


def kernel(e, x, feat_table, name_emb):
    B = e.shape[0]
    D = feat_table.shape[1]
    info = plsc.get_sparse_core_info()
    NC, NS = info.num_cores, info.num_subcores
    NW = NC * NS
    e2 = e.astype(jnp.int32).reshape(NW, B // NW)
    XW = x.shape[1]
    xf = x.astype(jnp.int32).reshape(x.shape[0] * XW)
    sc_kernel = _build(B, D, XW, NC, NS)
    return sc_kernel(e2, xf, feat_table, name_emb)


# speedup vs baseline: 11.3061x; 1.5661x over previous
import functools

import jax
import jax.numpy as jnp
from jax import lax
from jax.experimental import pallas as pl
from jax.experimental.pallas import tpu as pltpu
from jax.experimental.pallas import tpu_sc as plsc

NUM_USERS = 200000
LANES = 16
CHUNK = 128


@functools.cache
def _build(B, D, NC, NS):
    NW = NC * NS
    b_per_w = B // NW
    n_chunks = b_per_w // CHUNK
    n_groups = b_per_w // LANES
    n_dchunks = D // LANES
    zero_row = 68
    mesh = plsc.VectorSubcoreMesh(core_axis_name="c", subcore_axis_name="s")

    @functools.partial(
        pl.kernel,
        mesh=mesh,
        compiler_params=pltpu.CompilerParams(
            needs_layout_passes=False, use_tc_tiling_on_sc=False),
        out_type=jax.ShapeDtypeStruct((B, D), jnp.float32),
        scratch_types=[
            pltpu.VMEM((b_per_w,), jnp.int32),
            pltpu.VMEM((4, b_per_w), jnp.int32),
            pltpu.VMEM((b_per_w, D), jnp.float32),
            pltpu.VMEM((69, D), jnp.float32),
            pltpu.SemaphoreType.DMA,
        ],
    )
    def sc_kernel(e_hbm, x3_hbm, x4_hbm, x5_hbm, x6_hbm, ft_hbm, name_hbm,
                  out_hbm, e_v, xc_v, acc_v, ft_v, sem):
        wid = lax.axis_index("s") * NC + lax.axis_index("c")
        base = wid * b_per_w

        pltpu.sync_copy(e_hbm.at[wid], e_v)
        pltpu.sync_copy(ft_hbm, ft_v)

        descs = []
        for k in range(n_chunks):
            idx = e_v.at[pl.ds(k * CHUNK, CHUNK)]
            descs.append(pltpu.async_copy(
                name_hbm.at[idx], acc_v.at[pl.ds(k * CHUNK, CHUNK)], sem))
            for c, xh in enumerate((x3_hbm, x4_hbm, x5_hbm, x6_hbm)):
                descs.append(pltpu.async_copy(
                    xh.at[idx], xc_v.at[c, pl.ds(k * CHUNK, CHUNK)], sem))
        for dsc in descs:
            dsc.wait()

        def group(g, carry):
            gbase = g * LANES
            ev = e_v[pl.ds(gbase, LANES)]
            mask = ev >= NUM_USERS
            f3 = jnp.where(mask, xc_v[0, pl.ds(gbase, LANES)] + 46, zero_row)
            f4 = jnp.where(mask, xc_v[1, pl.ds(gbase, LANES)] + 35, zero_row)
            f5 = jnp.where(mask, xc_v[2, pl.ds(gbase, LANES)], zero_row)
            f6 = jnp.where(mask, xc_v[3, pl.ds(gbase, LANES)] + 9, zero_row)
            for l in range(LANES):
                r = gbase + l
                s3, s4, s5, s6 = f3[l], f4[l], f5[l], f6[l]
                for c in range(n_dchunks):
                    dcol = pl.ds(c * LANES, LANES)
                    acc_v[r, dcol] = (acc_v[r, dcol]
                                      + ft_v[s5, dcol] + ft_v[s6, dcol]
                                      + ft_v[s4, dcol] + ft_v[s3, dcol])
            return carry

        lax.fori_loop(0, n_groups, group, 0)
        pltpu.sync_copy(acc_v, out_hbm.at[pl.ds(base, b_per_w)])

    return sc_kernel


def kernel(e, x, feat_table, name_emb):
    B = e.shape[0]
    D = feat_table.shape[1]
    info = plsc.get_sparse_core_info()
    NC, NS = info.num_cores, info.num_subcores
    NW = NC * NS
    e2 = e.astype(jnp.int32).reshape(NW, B // NW)
    xi = x.astype(jnp.int32)
    x3, x4, x5, x6 = xi[:, 3], xi[:, 4], xi[:, 5], xi[:, 6]
    ftp = jnp.concatenate(
        [feat_table, jnp.zeros((1, D), feat_table.dtype)], axis=0)
    sc_kernel = _build(B, D, NC, NS)
    return sc_kernel(e2, x3, x4, x5, x6, ftp, name_emb)
